# baseline (speedup 1.0000x reference)
import functools

import jax
import jax.numpy as jnp
from jax import lax
from jax.experimental import pallas as pl
from jax.experimental.pallas import tpu as pltpu
from jax.experimental.pallas import tpu_sc as plsc

N = 10000
NP = 10240
E = 320000
IN_DIM = 128
HEADS = 4
OUT_DIM = 32
HD = HEADS * OUT_DIM
F = 136
NTILES = 32
ET = E // NTILES
C = 128
NCF = (ET // C)
TAIL = ET - NCF * C
HC = C // 2
STRIPE = NP // 16


def _stage1_body(x_ref, wt_ref, s_ref, f_ref, a_ref):
    proj = jnp.dot(x_ref[...], wt_ref[...], preferred_element_type=jnp.float32)
    al16 = jnp.dot(proj, s_ref[...], preferred_element_type=jnp.float32)
    f_ref[...] = jnp.concatenate([proj, al16[:, :8]], axis=1)
    a_ref[...] = al16[:, :8]


def _stage1(x, W, attn_src, attn_dst):
    n = x.shape[0]
    blk = 512
    eye = jnp.eye(HEADS, dtype=jnp.float32)
    s_src = (eye[:, None, :] * attn_src[:, :, None]).reshape(HD, HEADS)
    s_dst = (eye[:, None, :] * attn_dst[:, :, None]).reshape(HD, HEADS)
    S = jnp.concatenate(
        [s_src, s_dst, jnp.zeros((HD, 8), jnp.float32)], axis=1)
    return pl.pallas_call(
        _stage1_body,
        grid=(n // blk,),
        in_specs=[
            pl.BlockSpec((blk, IN_DIM), lambda i: (i, 0)),
            pl.BlockSpec((IN_DIM, HD), lambda i: (0, 0)),
            pl.BlockSpec((HD, 16), lambda i: (0, 0)),
        ],
        out_specs=[
            pl.BlockSpec((blk, F), lambda i: (i, 0)),
            pl.BlockSpec((blk, 8), lambda i: (i, 0)),
        ],
        out_shape=[
            jax.ShapeDtypeStruct((n, F), jnp.float32),
            jax.ShapeDtypeStruct((n, 8), jnp.float32),
        ],
    )(x, W.T, S)


def _sc_edge_body(ftab, u_hbm, v_hbm, zeros_hbm, out_hbm,
                  u_v0, v_v0, u_v1, v_v1, u_t, v_t, rows_a, rows_b,
                  tail_a, tail_b, acc, sem_a, sem_b, sem_a1, sem_b1,
                  ss_a, ss_b):
    c = lax.axis_index("c")
    s = lax.axis_index("s")
    wid = s * 2 + c
    iota = lax.iota(jnp.int32, 16)

    pltpu.sync_copy(zeros_hbm, acc.at[pl.ds(s * STRIPE, STRIPE)])
    plsc.subcore_barrier()

    def full(val):
        return jnp.full((16,), val, jnp.int32)

    def _edge_groups(ra, rb, iota, g0, ngroups):
        def group_body(gg, carry2):
            g = g0 + gg
            ridx = g * 16 + iota
            a_su = [plsc.load_gather(ra, [ridx, full(128 + h)])
                    for h in range(HEADS)]
            a_du = [plsc.load_gather(ra, [ridx, full(132 + h)])
                    for h in range(HEADS)]
            a_sv = [plsc.load_gather(rb, [ridx, full(128 + h)])
                    for h in range(HEADS)]
            a_dv = [plsc.load_gather(rb, [ridx, full(132 + h)])
                    for h in range(HEADS)]

            def lrelu(t):
                return jnp.maximum(t, 0.2 * t)

            w1 = [jnp.exp(lrelu(a_su[h] + a_dv[h])) for h in range(HEADS)]
            w2 = [jnp.exp(lrelu(a_sv[h] + a_du[h])) for h in range(HEADS)]
            for h in range(HEADS):
                plsc.store_scatter(ra, [ridx, full(128 + h)], w1[h])
                plsc.store_scatter(rb, [ridx, full(128 + h)], w2[h])
            for k in range(16):
                e = g * 16 + k
                for rows, wt in ((ra, w1), (rb, w2)):
                    wv = [jnp.full((16,), wt[h][k]) for h in range(HEADS)]
                    for cb in range(8):
                        sl = pl.ds(cb * 16, 16)
                        rows[e, sl] = rows[e, sl] * wv[cb // 2]
            return carry2

        lax.fori_loop(0, ngroups, group_body, 0)

    def chunk_work(j, u_v, v_v, first=False):
        eb = wid * ET + j * C
        pltpu.sync_copy(u_hbm.at[pl.ds(eb, C)], u_v)
        pltpu.sync_copy(v_hbm.at[pl.ds(eb, C)], v_v)

        def _wait_prev_scatter():
            pltpu.make_async_copy(rows_a, acc.at[v_v], ss_a).wait()
            pltpu.make_async_copy(rows_b, acc.at[u_v], ss_b).wait()

        if first:
            pl.when(j > 0)(_wait_prev_scatter)
        else:
            _wait_prev_scatter()

        h0 = pl.ds(0, HC)
        h1 = pl.ds(HC, HC)
        ca0 = pltpu.async_copy(ftab.at[u_v.at[h0]], rows_a.at[h0], sem_a)
        cb0 = pltpu.async_copy(ftab.at[v_v.at[h0]], rows_b.at[h0], sem_b)
        ca1 = pltpu.async_copy(ftab.at[u_v.at[h1]], rows_a.at[h1], sem_a1)
        cb1 = pltpu.async_copy(ftab.at[v_v.at[h1]], rows_b.at[h1], sem_b1)
        ca0.wait()
        cb0.wait()
        _edge_groups(rows_a, rows_b, iota, 0, HC // 16)
        ca1.wait()
        cb1.wait()
        _edge_groups(rows_a, rows_b, iota, HC // 16, HC // 16)
        pltpu.async_copy(rows_a, acc.at[v_v], ss_a, add=True)
        pltpu.async_copy(rows_b, acc.at[u_v], ss_b, add=True)

    def chunk_body(i, carry):
        chunk_work(2 * i, u_v0, v_v0, first=True)
        chunk_work(2 * i + 1, u_v1, v_v1)
        return carry

    lax.fori_loop(0, NCF // 2, chunk_body, 0)

    tb = wid * ET + NCF * C
    pltpu.sync_copy(u_hbm.at[pl.ds(tb, TAIL)], u_t)
    pltpu.sync_copy(v_hbm.at[pl.ds(tb, TAIL)], v_t)
    cp_a = pltpu.async_copy(ftab.at[u_t], tail_a, sem_a)
    cp_b = pltpu.async_copy(ftab.at[v_t], tail_b, sem_b)
    cp_a.wait()
    cp_b.wait()
    _edge_groups(tail_a, tail_b, iota, 0, 1)
    pltpu.make_async_copy(rows_a, acc.at[v_v0], ss_a).wait()
    pltpu.make_async_copy(rows_b, acc.at[u_v0], ss_b).wait()
    pltpu.async_copy(tail_a, acc.at[v_t], ss_a, add=True)
    pltpu.async_copy(tail_b, acc.at[u_t], ss_b, add=True)
    pltpu.make_async_copy(tail_a, acc.at[v_t], ss_a).wait()
    pltpu.make_async_copy(tail_b, acc.at[u_t], ss_b).wait()
    plsc.subcore_barrier()
    pltpu.sync_copy(acc.at[pl.ds(s * STRIPE, STRIPE)],
                    out_hbm.at[c, pl.ds(s * STRIPE, STRIPE)])


def _sc_edge(ftab, u_idx, v_idx):
    zeros = jnp.zeros((STRIPE, F), jnp.float32)
    mesh = plsc.VectorSubcoreMesh(core_axis_name="c", subcore_axis_name="s")
    return pl.kernel(
        _sc_edge_body,
        out_type=jax.ShapeDtypeStruct((2, NP, F), jnp.float32),
        mesh=mesh,
        compiler_params=pltpu.CompilerParams(use_tc_tiling_on_sc=False,
                                             needs_layout_passes=False),
        scratch_types=[
            pltpu.VMEM((C,), jnp.int32),
            pltpu.VMEM((C,), jnp.int32),
            pltpu.VMEM((C,), jnp.int32),
            pltpu.VMEM((C,), jnp.int32),
            pltpu.VMEM((TAIL,), jnp.int32),
            pltpu.VMEM((TAIL,), jnp.int32),
            pltpu.VMEM((C, F), jnp.float32),
            pltpu.VMEM((C, F), jnp.float32),
            pltpu.VMEM((TAIL, F), jnp.float32),
            pltpu.VMEM((TAIL, F), jnp.float32),
            pltpu.VMEM_SHARED((NP, F), jnp.float32),
            pltpu.SemaphoreType.DMA,
            pltpu.SemaphoreType.DMA,
            pltpu.SemaphoreType.DMA,
            pltpu.SemaphoreType.DMA,
            pltpu.SemaphoreType.DMA,
            pltpu.SemaphoreType.DMA,
        ],
    )(ftab, u_idx, v_idx, zeros)


def _stage3_body(f_ref, a_ref, p_ref, b_ref, o_ref):
    al = a_ref[...]
    a_s = al[:, 0:4]
    a_d = al[:, 4:8]
    sc = a_s + a_d
    w_self = jnp.exp(jnp.maximum(sc, 0.2 * sc))
    p = p_ref[...]
    acc = p[0] + p[1]
    outs = []
    for h in range(HEADS):
        lo = h * OUT_DIM
        num = (acc[:, lo:lo + OUT_DIM]
               + f_ref[:, lo:lo + OUT_DIM] * w_self[:, h:h + 1])
        den = jnp.clip(acc[:, 128 + h:129 + h] + w_self[:, h:h + 1],
                       1e-12, None)
        outs.append(num / den)
    o = jnp.concatenate(outs, axis=1) + b_ref[...]
    o_ref[...] = jnp.where(o > 0, o, jnp.exp(jnp.minimum(o, 0.0)) - 1.0)


def _stage3(ftab, alph, parts, bias2):
    blk = 400
    return pl.pallas_call(
        _stage3_body,
        grid=(N // blk,),
        in_specs=[
            pl.BlockSpec((blk, F), lambda i: (i, 0)),
            pl.BlockSpec((blk, 8), lambda i: (i, 0)),
            pl.BlockSpec((2, blk, F), lambda i: (0, i, 0)),
            pl.BlockSpec((1, HD), lambda i: (0, 0)),
        ],
        out_specs=pl.BlockSpec((blk, HD), lambda i: (i, 0)),
        out_shape=jax.ShapeDtypeStruct((N, HD), jnp.float32),
    )(ftab, alph, parts, bias2)


def kernel(x, edge_index, num_nodes, W, attn_src, attn_dst, bias):
    n = x.shape[0]
    xp = jnp.pad(x, ((0, NP - n), (0, 0)))
    ftab, alph = _stage1(xp, W, attn_src, attn_dst)
    parts = _sc_edge(ftab, edge_index[0], edge_index[1])
    delta = (jnp.asarray(num_nodes) - n).astype(jnp.float32)
    bias2 = (bias + delta).reshape(1, HD)
    return _stage3(ftab, alph, parts, bias2)


# speedup vs baseline: 1.0044x; 1.0044x over previous
import functools

import jax
import jax.numpy as jnp
from jax import lax
from jax.experimental import pallas as pl
from jax.experimental.pallas import tpu as pltpu
from jax.experimental.pallas import tpu_sc as plsc

N = 10000
NP = 10240
E = 320000
IN_DIM = 128
HEADS = 4
OUT_DIM = 32
HD = HEADS * OUT_DIM
F = 136
NTILES = 32
ET = E // NTILES
C = 128
NCF = (ET // C)
TAIL = ET - NCF * C
HC = C // 2
STRIPE = NP // 16


def _stage1_body(x_ref, wt_ref, s_ref, f_ref, a_ref):
    proj = jnp.dot(x_ref[...], wt_ref[...], preferred_element_type=jnp.float32)
    al16 = jnp.dot(proj, s_ref[...], preferred_element_type=jnp.float32)
    f_ref[...] = jnp.concatenate([proj, al16[:, :8]], axis=1)
    a_ref[...] = al16[:, :8]


def _stage1(x, W, attn_src, attn_dst):
    n = x.shape[0]
    blk = 400
    eye = jnp.eye(HEADS, dtype=jnp.float32)
    s_src = (eye[:, None, :] * attn_src[:, :, None]).reshape(HD, HEADS)
    s_dst = (eye[:, None, :] * attn_dst[:, :, None]).reshape(HD, HEADS)
    S = jnp.concatenate(
        [s_src, s_dst, jnp.zeros((HD, 8), jnp.float32)], axis=1)
    return pl.pallas_call(
        _stage1_body,
        grid=(n // blk,),
        in_specs=[
            pl.BlockSpec((blk, IN_DIM), lambda i: (i, 0)),
            pl.BlockSpec((IN_DIM, HD), lambda i: (0, 0)),
            pl.BlockSpec((HD, 16), lambda i: (0, 0)),
        ],
        out_specs=[
            pl.BlockSpec((blk, F), lambda i: (i, 0)),
            pl.BlockSpec((blk, 8), lambda i: (i, 0)),
        ],
        out_shape=[
            jax.ShapeDtypeStruct((NP, F), jnp.float32),
            jax.ShapeDtypeStruct((NP, 8), jnp.float32),
        ],
    )(x, W.T, S)


def _sc_edge_body(ftab, u_hbm, v_hbm, zeros_hbm, out_hbm,
                  u_v0, v_v0, u_v1, v_v1, u_t, v_t, rows_a, rows_b,
                  tail_a, tail_b, acc, sem_a, sem_b, sem_a1, sem_b1,
                  ss_a, ss_b, si0, si1):
    c = lax.axis_index("c")
    s = lax.axis_index("s")
    wid = s * 2 + c
    iota = lax.iota(jnp.int32, 16)

    pltpu.sync_copy(zeros_hbm, acc.at[pl.ds(s * STRIPE, STRIPE)])
    plsc.subcore_barrier()

    def full(val):
        return jnp.full((16,), val, jnp.int32)

    def _edge_groups(ra, rb, iota, g0, ngroups):
        def group_body(gg, carry2):
            g = g0 + gg
            ridx = g * 16 + iota
            a_su = [plsc.load_gather(ra, [ridx, full(128 + h)])
                    for h in range(HEADS)]
            a_du = [plsc.load_gather(ra, [ridx, full(132 + h)])
                    for h in range(HEADS)]
            a_sv = [plsc.load_gather(rb, [ridx, full(128 + h)])
                    for h in range(HEADS)]
            a_dv = [plsc.load_gather(rb, [ridx, full(132 + h)])
                    for h in range(HEADS)]

            def lrelu(t):
                return jnp.maximum(t, 0.2 * t)

            w1 = [jnp.exp(lrelu(a_su[h] + a_dv[h])) for h in range(HEADS)]
            w2 = [jnp.exp(lrelu(a_sv[h] + a_du[h])) for h in range(HEADS)]
            for h in range(HEADS):
                plsc.store_scatter(ra, [ridx, full(128 + h)], w1[h])
                plsc.store_scatter(rb, [ridx, full(128 + h)], w2[h])
            for k in range(16):
                e = g * 16 + k
                for rows, wt in ((ra, w1), (rb, w2)):
                    wv = [jnp.full((16,), wt[h][k]) for h in range(HEADS)]
                    for cb in range(8):
                        sl = pl.ds(cb * 16, 16)
                        rows[e, sl] = rows[e, sl] * wv[cb // 2]
            return carry2

        lax.fori_loop(0, ngroups, group_body, 0)

    def issue_idx(j, u_v, v_v, si):
        eb = wid * ET + j * C
        pltpu.async_copy(u_hbm.at[pl.ds(eb, C)], u_v, si)
        pltpu.async_copy(v_hbm.at[pl.ds(eb, C)], v_v, si)

    def chunk_work(j, u_v, v_v, si, nxt, first=False):
        pltpu.make_async_copy(u_hbm.at[pl.ds(0, C)], u_v, si).wait()
        pltpu.make_async_copy(v_hbm.at[pl.ds(0, C)], v_v, si).wait()

        def _wait_prev_scatter():
            pltpu.make_async_copy(rows_a, acc.at[v_v], ss_a).wait()
            pltpu.make_async_copy(rows_b, acc.at[u_v], ss_b).wait()

        if first:
            pl.when(j > 0)(_wait_prev_scatter)
        else:
            _wait_prev_scatter()

        if nxt is not None:
            nu_v, nv_v, nsi = nxt

            @pl.when(j + 1 <= NCF - 1)
            def _prefetch_idx():
                issue_idx(j + 1, nu_v, nv_v, nsi)

        h0 = pl.ds(0, HC)
        h1 = pl.ds(HC, HC)
        ca0 = pltpu.async_copy(ftab.at[u_v.at[h0]], rows_a.at[h0], sem_a)
        cb0 = pltpu.async_copy(ftab.at[v_v.at[h0]], rows_b.at[h0], sem_b)
        ca1 = pltpu.async_copy(ftab.at[u_v.at[h1]], rows_a.at[h1], sem_a1)
        cb1 = pltpu.async_copy(ftab.at[v_v.at[h1]], rows_b.at[h1], sem_b1)
        ca0.wait()
        cb0.wait()
        _edge_groups(rows_a, rows_b, iota, 0, HC // 16)
        ca1.wait()
        cb1.wait()
        _edge_groups(rows_a, rows_b, iota, HC // 16, HC // 16)
        pltpu.async_copy(rows_a, acc.at[v_v], ss_a, add=True)
        pltpu.async_copy(rows_b, acc.at[u_v], ss_b, add=True)

    def chunk_body(i, carry):
        chunk_work(2 * i, u_v0, v_v0, si0, (u_v1, v_v1, si1), first=True)
        chunk_work(2 * i + 1, u_v1, v_v1, si1, (u_v0, v_v0, si0))
        return carry

    issue_idx(jnp.int32(0), u_v0, v_v0, si0)
    lax.fori_loop(0, NCF // 2, chunk_body, 0)

    tb = wid * ET + NCF * C
    pltpu.sync_copy(u_hbm.at[pl.ds(tb, TAIL)], u_t)
    pltpu.sync_copy(v_hbm.at[pl.ds(tb, TAIL)], v_t)
    cp_a = pltpu.async_copy(ftab.at[u_t], tail_a, sem_a)
    cp_b = pltpu.async_copy(ftab.at[v_t], tail_b, sem_b)
    cp_a.wait()
    cp_b.wait()
    _edge_groups(tail_a, tail_b, iota, 0, 1)
    pltpu.make_async_copy(rows_a, acc.at[v_v0], ss_a).wait()
    pltpu.make_async_copy(rows_b, acc.at[u_v0], ss_b).wait()
    pltpu.async_copy(tail_a, acc.at[v_t], ss_a, add=True)
    pltpu.async_copy(tail_b, acc.at[u_t], ss_b, add=True)
    pltpu.make_async_copy(tail_a, acc.at[v_t], ss_a).wait()
    pltpu.make_async_copy(tail_b, acc.at[u_t], ss_b).wait()
    plsc.subcore_barrier()
    pltpu.sync_copy(acc.at[pl.ds(s * STRIPE, STRIPE)],
                    out_hbm.at[c, pl.ds(s * STRIPE, STRIPE)])


def _sc_edge(ftab, u_idx, v_idx):
    zeros = jnp.zeros((STRIPE, F), jnp.float32)
    mesh = plsc.VectorSubcoreMesh(core_axis_name="c", subcore_axis_name="s")
    return pl.kernel(
        _sc_edge_body,
        out_type=jax.ShapeDtypeStruct((2, NP, F), jnp.float32),
        mesh=mesh,
        compiler_params=pltpu.CompilerParams(use_tc_tiling_on_sc=False,
                                             needs_layout_passes=False),
        scratch_types=[
            pltpu.VMEM((C,), jnp.int32),
            pltpu.VMEM((C,), jnp.int32),
            pltpu.VMEM((C,), jnp.int32),
            pltpu.VMEM((C,), jnp.int32),
            pltpu.VMEM((TAIL,), jnp.int32),
            pltpu.VMEM((TAIL,), jnp.int32),
            pltpu.VMEM((C, F), jnp.float32),
            pltpu.VMEM((C, F), jnp.float32),
            pltpu.VMEM((TAIL, F), jnp.float32),
            pltpu.VMEM((TAIL, F), jnp.float32),
            pltpu.VMEM_SHARED((NP, F), jnp.float32),
            pltpu.SemaphoreType.DMA,
            pltpu.SemaphoreType.DMA,
            pltpu.SemaphoreType.DMA,
            pltpu.SemaphoreType.DMA,
            pltpu.SemaphoreType.DMA,
            pltpu.SemaphoreType.DMA,
            pltpu.SemaphoreType.DMA,
            pltpu.SemaphoreType.DMA,
        ],
    )(ftab, u_idx, v_idx, zeros)


def _stage3_body(f_ref, a_ref, p_ref, b_ref, o_ref):
    al = a_ref[...]
    a_s = al[:, 0:4]
    a_d = al[:, 4:8]
    sc = a_s + a_d
    w_self = jnp.exp(jnp.maximum(sc, 0.2 * sc))
    p = p_ref[...]
    acc = p[0] + p[1]
    outs = []
    for h in range(HEADS):
        lo = h * OUT_DIM
        num = (acc[:, lo:lo + OUT_DIM]
               + f_ref[:, lo:lo + OUT_DIM] * w_self[:, h:h + 1])
        den = jnp.clip(acc[:, 128 + h:129 + h] + w_self[:, h:h + 1],
                       1e-12, None)
        outs.append(num / den)
    o = jnp.concatenate(outs, axis=1) + b_ref[...]
    o_ref[...] = jnp.where(o > 0, o, jnp.exp(jnp.minimum(o, 0.0)) - 1.0)


def _stage3(ftab, alph, parts, bias2):
    blk = 400
    return pl.pallas_call(
        _stage3_body,
        grid=(N // blk,),
        in_specs=[
            pl.BlockSpec((blk, F), lambda i: (i, 0)),
            pl.BlockSpec((blk, 8), lambda i: (i, 0)),
            pl.BlockSpec((2, blk, F), lambda i: (0, i, 0)),
            pl.BlockSpec((1, HD), lambda i: (0, 0)),
        ],
        out_specs=pl.BlockSpec((blk, HD), lambda i: (i, 0)),
        out_shape=jax.ShapeDtypeStruct((N, HD), jnp.float32),
    )(ftab, alph, parts, bias2)


def kernel(x, edge_index, num_nodes, W, attn_src, attn_dst, bias):
    n = x.shape[0]
    ftab, alph = _stage1(x, W, attn_src, attn_dst)
    parts = _sc_edge(ftab, edge_index[0], edge_index[1])
    delta = (jnp.asarray(num_nodes) - n).astype(jnp.float32)
    bias2 = (bias + delta).reshape(1, HD)
    return _stage3(ftab, alph, parts, bias2)
